# baseline (device time: 32508 ns/iter reference)
import jax
import jax.numpy as jnp
from jax import lax
from jax.experimental import pallas as pl
from jax.experimental.pallas import tpu as pltpu

N_LAYERS = 3
C = 8
D = 2


def kernel(x, Win0, Wout0, Win1, Wout1, Win2, Wout2):
    b, d_in = x.shape
    _, h_per = Win0.shape
    _, o_per = Wout0.shape
    hc = h_per // C
    oc = o_per // D

    def body(
        x_ref, win0_ref, wout0_ref, win1_ref, wout1_ref, win2_ref, wout2_ref,
        out_ref,
        win_scr, wout_scr,
        hsend_bf, hrecv_bf, osend_bf, orecv_bf,
        w_sems, w0_sems,
        hs_sems, hr_sems, os_sems, or_sems,
    ):
        my_x = lax.axis_index("x")
        my_y = lax.axis_index("y")
        y_peer = (my_x, 1 - my_y)
        x_peer = (1 - my_x, my_y)

        win_hbm = [win0_ref, win1_ref, win2_ref]
        wout_hbm = [wout0_ref, wout1_ref, wout2_ref]
        win0_copies = [
            pltpu.make_async_copy(
                win_hbm[0].at[:, c * hc:(c + 1) * hc],
                win_scr.at[0, :, c * hc:(c + 1) * hc],
                w0_sems.at[c],
            )
            for c in range(C)
        ]
        w_copies = []
        for l in range(N_LAYERS):
            cin = pltpu.make_async_copy(
                win_hbm[l], win_scr.at[l], w_sems.at[2 * l]
            )
            cout = pltpu.make_async_copy(
                wout_hbm[l], wout_scr.at[l], w_sems.at[2 * l + 1]
            )
            w_copies.append((cin, cout))
        for cp in win0_copies:
            cp.start()
        w_copies[0][1].start()
        for cin, cout in w_copies[1:]:
            cin.start()
            cout.start()

        barrier = pltpu.get_barrier_semaphore()
        for nbr in (y_peer, x_peer):
            pl.semaphore_signal(
                barrier, inc=1, device_id=nbr,
                device_id_type=pl.DeviceIdType.MESH,
            )

        h_rdmas = {}
        o_rdmas = {}

        def h_send(l, c):
            r = pltpu.make_async_remote_copy(
                src_ref=hsend_bf.at[l, c],
                dst_ref=hrecv_bf.at[l, c],
                send_sem=hs_sems.at[l, c],
                recv_sem=hr_sems.at[l, c],
                device_id=y_peer,
                device_id_type=pl.DeviceIdType.MESH,
            )
            h_rdmas[(l, c)] = r
            r.start()

        def o_send(l, d):
            r = pltpu.make_async_remote_copy(
                src_ref=osend_bf.at[l, d],
                dst_ref=orecv_bf.at[l, d],
                send_sem=os_sems.at[l, d],
                recv_sem=or_sems.at[l, d],
                device_id=x_peer,
                device_id_type=pl.DeviceIdType.MESH,
            )
            o_rdmas[(l, d)] = r
            r.start()

        def dot(a, b_val):
            return jnp.dot(a, b_val, preferred_element_type=jnp.float32)

        cur = x_ref[...]
        hpart_vals = [None] * C
        for c in range(C):
            win0_copies[c].wait()
            val = dot(cur, win_scr[0, :, c * hc:(c + 1) * hc])
            hpart_vals[c] = val
            hsend_bf[0, c] = val.astype(jnp.bfloat16)
            if c == 0:
                pl.semaphore_wait(barrier, 2)
            h_send(0, c)

        for l in range(N_LAYERS):
            w_copies[l][1].wait()
            oaccs = [None] * D
            for c in range(C):
                h_rdmas[(l, c)].wait_recv()
                hval = jnp.maximum(
                    hpart_vals[c] + hrecv_bf[l, c].astype(jnp.float32), 0.0
                )
                for d in range(D):
                    q = dot(
                        hval,
                        wout_scr[l, c * hc:(c + 1) * hc, d * oc:(d + 1) * oc],
                    )
                    oaccs[d] = q if oaccs[d] is None else oaccs[d] + q
                    if c == C - 1:
                        osend_bf[l, d] = oaccs[d].astype(jnp.bfloat16)
                        o_send(l, d)

            if l < N_LAYERS - 1:
                w_copies[l + 1][0].wait()
                g = [None] * C
                hpart_vals = g
                for d in range(D):
                    o_rdmas[(l, d)].wait_recv()
                    cur_d = oaccs[d] + orecv_bf[l, d].astype(jnp.float32)
                    for c in range(C):
                        q = dot(
                            cur_d,
                            win_scr[
                                l + 1,
                                d * oc:(d + 1) * oc,
                                c * hc:(c + 1) * hc,
                            ],
                        )
                        g[c] = q if g[c] is None else g[c] + q
                        if d == D - 1:
                            hsend_bf[l + 1, c] = g[c].astype(jnp.bfloat16)
                            h_send(l + 1, c)
            else:
                for d in range(D):
                    o_rdmas[(l, d)].wait_recv()
                    out_ref[:, d * oc:(d + 1) * oc] = (
                        oaccs[d] + orecv_bf[l, d].astype(jnp.float32)
                    )

        for r in h_rdmas.values():
            r.wait_send()
        for r in o_rdmas.values():
            r.wait_send()

    return pl.pallas_call(
        body,
        out_shape=jax.ShapeDtypeStruct((b, o_per), jnp.float32),
        in_specs=[pl.BlockSpec(memory_space=pltpu.VMEM)]
        + [pl.BlockSpec(memory_space=pl.ANY)] * 6,
        out_specs=pl.BlockSpec(memory_space=pltpu.VMEM),
        scratch_shapes=[
            pltpu.VMEM((N_LAYERS, d_in, h_per), jnp.float32),
            pltpu.VMEM((N_LAYERS, h_per, o_per), jnp.float32),
            pltpu.VMEM((N_LAYERS, C, b, hc), jnp.bfloat16),
            pltpu.VMEM((N_LAYERS, C, b, hc), jnp.bfloat16),
            pltpu.VMEM((N_LAYERS, D, b, oc), jnp.bfloat16),
            pltpu.VMEM((N_LAYERS, D, b, oc), jnp.bfloat16),
            pltpu.SemaphoreType.DMA((2 * N_LAYERS,)),
            pltpu.SemaphoreType.DMA((C,)),
            pltpu.SemaphoreType.DMA((N_LAYERS, C)),
            pltpu.SemaphoreType.DMA((N_LAYERS, C)),
            pltpu.SemaphoreType.DMA((N_LAYERS, D)),
            pltpu.SemaphoreType.DMA((N_LAYERS, D)),
        ],
        compiler_params=pltpu.CompilerParams(
            collective_id=0,
            vmem_limit_bytes=100 * 1024 * 1024,
        ),
    )(x, Win0, Wout0, Win1, Wout1, Win2, Wout2)


# device time: 32342 ns/iter; 1.0051x vs baseline; 1.0051x over previous
import jax
import jax.numpy as jnp
from jax import lax
from jax.experimental import pallas as pl
from jax.experimental.pallas import tpu as pltpu

N_LAYERS = 3
C = 4
D = 2


def kernel(x, Win0, Wout0, Win1, Wout1, Win2, Wout2):
    b, d_in = x.shape
    _, h_per = Win0.shape
    _, o_per = Wout0.shape
    hc = h_per // C
    oc = o_per // D

    def body(
        x_ref, win0_ref, wout0_ref, win1_ref, wout1_ref, win2_ref, wout2_ref,
        out_ref,
        win_scr, wout_scr,
        hsend_bf, hrecv_bf, osend_bf, orecv_bf,
        w_sems, w0_sems,
        hs_sems, hr_sems, os_sems, or_sems,
    ):
        my_x = lax.axis_index("x")
        my_y = lax.axis_index("y")
        y_peer = (my_x, 1 - my_y)
        x_peer = (1 - my_x, my_y)

        win_hbm = [win0_ref, win1_ref, win2_ref]
        wout_hbm = [wout0_ref, wout1_ref, wout2_ref]
        win0_copies = [
            pltpu.make_async_copy(
                win_hbm[0].at[:, c * hc:(c + 1) * hc],
                win_scr.at[0, :, c * hc:(c + 1) * hc],
                w0_sems.at[c],
            )
            for c in range(C)
        ]
        w_copies = [(None,
                     pltpu.make_async_copy(
                         wout_hbm[0], wout_scr.at[0], w_sems.at[1]))]
        for l in range(1, N_LAYERS):
            w_copies.append((
                pltpu.make_async_copy(
                    win_hbm[l], win_scr.at[l], w_sems.at[2 * l]),
                pltpu.make_async_copy(
                    wout_hbm[l], wout_scr.at[l], w_sems.at[2 * l + 1]),
            ))
        for cp in win0_copies:
            cp.start()
        w_copies[0][1].start()
        for cin, cout in w_copies[1:]:
            cin.start()
            cout.start()

        barrier = pltpu.get_barrier_semaphore()
        for nbr in (y_peer, x_peer):
            pl.semaphore_signal(
                barrier, inc=1, device_id=nbr,
                device_id_type=pl.DeviceIdType.MESH,
            )

        h_rdmas = {}
        o_rdmas = {}

        def h_send(l, c):
            r = pltpu.make_async_remote_copy(
                src_ref=hsend_bf.at[l, c],
                dst_ref=hrecv_bf.at[l, c],
                send_sem=hs_sems.at[l, c],
                recv_sem=hr_sems.at[l, c],
                device_id=y_peer,
                device_id_type=pl.DeviceIdType.MESH,
            )
            h_rdmas[(l, c)] = r
            r.start()

        def o_send(l, d):
            r = pltpu.make_async_remote_copy(
                src_ref=osend_bf.at[l, d],
                dst_ref=orecv_bf.at[l, d],
                send_sem=os_sems.at[l, d],
                recv_sem=or_sems.at[l, d],
                device_id=x_peer,
                device_id_type=pl.DeviceIdType.MESH,
            )
            o_rdmas[(l, d)] = r
            r.start()

        def dot(a, b_val):
            return jnp.dot(a, b_val, preferred_element_type=jnp.float32)

        cur = x_ref[...]
        hpart_vals = [None] * C
        for c in range(C):
            win0_copies[c].wait()
            val = dot(cur, win_scr[0, :, c * hc:(c + 1) * hc])
            hpart_vals[c] = val
            hsend_bf[0, c] = val.astype(jnp.bfloat16)
            if c == 0:
                pl.semaphore_wait(barrier, 2)
            h_send(0, c)

        for l in range(N_LAYERS):
            w_copies[l][1].wait()
            oaccs = [None] * D
            for c in range(C):
                h_rdmas[(l, c)].wait_recv()
                hval = jnp.maximum(
                    hpart_vals[c] + hrecv_bf[l, c].astype(jnp.float32), 0.0
                )
                for d in range(D):
                    q = dot(
                        hval,
                        wout_scr[l, c * hc:(c + 1) * hc, d * oc:(d + 1) * oc],
                    )
                    oaccs[d] = q if oaccs[d] is None else oaccs[d] + q
                    if c == C - 1:
                        osend_bf[l, d] = oaccs[d].astype(jnp.bfloat16)
                        o_send(l, d)

            if l < N_LAYERS - 1:
                w_copies[l + 1][0].wait()
                g = [None] * C
                hpart_vals = g
                for d in range(D):
                    o_rdmas[(l, d)].wait_recv()
                    cur_d = oaccs[d] + orecv_bf[l, d].astype(jnp.float32)
                    for c in range(C):
                        q = dot(
                            cur_d,
                            win_scr[
                                l + 1,
                                d * oc:(d + 1) * oc,
                                c * hc:(c + 1) * hc,
                            ],
                        )
                        g[c] = q if g[c] is None else g[c] + q
                        if d == D - 1:
                            hsend_bf[l + 1, c] = g[c].astype(jnp.bfloat16)
                            h_send(l + 1, c)
            else:
                for d in range(D):
                    o_rdmas[(l, d)].wait_recv()
                    out_ref[:, d * oc:(d + 1) * oc] = (
                        oaccs[d] + orecv_bf[l, d].astype(jnp.float32)
                    )

        for r in h_rdmas.values():
            r.wait_send()
        for r in o_rdmas.values():
            r.wait_send()

    return pl.pallas_call(
        body,
        out_shape=jax.ShapeDtypeStruct((b, o_per), jnp.float32),
        in_specs=[pl.BlockSpec(memory_space=pltpu.VMEM)]
        + [pl.BlockSpec(memory_space=pl.ANY)] * 6,
        out_specs=pl.BlockSpec(memory_space=pltpu.VMEM),
        scratch_shapes=[
            pltpu.VMEM((N_LAYERS, d_in, h_per), jnp.float32),
            pltpu.VMEM((N_LAYERS, h_per, o_per), jnp.float32),
            pltpu.VMEM((N_LAYERS, C, b, hc), jnp.bfloat16),
            pltpu.VMEM((N_LAYERS, C, b, hc), jnp.bfloat16),
            pltpu.VMEM((N_LAYERS, D, b, oc), jnp.bfloat16),
            pltpu.VMEM((N_LAYERS, D, b, oc), jnp.bfloat16),
            pltpu.SemaphoreType.DMA((2 * N_LAYERS,)),
            pltpu.SemaphoreType.DMA((C,)),
            pltpu.SemaphoreType.DMA((N_LAYERS, C)),
            pltpu.SemaphoreType.DMA((N_LAYERS, C)),
            pltpu.SemaphoreType.DMA((N_LAYERS, D)),
            pltpu.SemaphoreType.DMA((N_LAYERS, D)),
        ],
        compiler_params=pltpu.CompilerParams(
            collective_id=0,
            vmem_limit_bytes=100 * 1024 * 1024,
        ),
    )(x, Win0, Wout0, Win1, Wout1, Win2, Wout2)


# device time: 29498 ns/iter; 1.1020x vs baseline; 1.0964x over previous
import jax
import jax.numpy as jnp
from jax import lax
from jax.experimental import pallas as pl
from jax.experimental.pallas import tpu as pltpu

N_LAYERS = 3
S = 2
C = 4
D = 2


def kernel(x, Win0, Wout0, Win1, Wout1, Win2, Wout2):
    b, d_in = x.shape
    _, h_per = Win0.shape
    _, o_per = Wout0.shape
    bs = b // S
    hc = h_per // C
    oc = o_per // D

    def body(
        x_ref, win0_ref, wout0_ref, win1_ref, wout1_ref, win2_ref, wout2_ref,
        out_ref,
        win_scr, wout_scr,
        hsend_bf, hrecv_bf, osend_bf, orecv_bf,
        w_sems, w0_sems,
        hs_sems, hr_sems, os_sems, or_sems,
    ):
        my_x = lax.axis_index("x")
        my_y = lax.axis_index("y")
        y_peer = (my_x, 1 - my_y)
        x_peer = (1 - my_x, my_y)

        win_hbm = [win0_ref, win1_ref, win2_ref]
        wout_hbm = [wout0_ref, wout1_ref, wout2_ref]
        win0_copies = [
            pltpu.make_async_copy(
                win_hbm[0].at[:, c * hc:(c + 1) * hc],
                win_scr.at[0, :, c * hc:(c + 1) * hc],
                w0_sems.at[c],
            )
            for c in range(C)
        ]
        w_copies = [(None,
                     pltpu.make_async_copy(
                         wout_hbm[0], wout_scr.at[0], w_sems.at[1]))]
        for l in range(1, N_LAYERS):
            w_copies.append((
                pltpu.make_async_copy(
                    win_hbm[l], win_scr.at[l], w_sems.at[2 * l]),
                pltpu.make_async_copy(
                    wout_hbm[l], wout_scr.at[l], w_sems.at[2 * l + 1]),
            ))
        for cp in win0_copies:
            cp.start()
        w_copies[0][1].start()
        for cin, cout in w_copies[1:]:
            cin.start()
            cout.start()

        barrier = pltpu.get_barrier_semaphore()
        for nbr in (y_peer, x_peer):
            pl.semaphore_signal(
                barrier, inc=1, device_id=nbr,
                device_id_type=pl.DeviceIdType.MESH,
            )

        h_rdmas = {}
        o_rdmas = {}

        def h_send(s, l, c):
            r = pltpu.make_async_remote_copy(
                src_ref=hsend_bf.at[s, l, c],
                dst_ref=hrecv_bf.at[s, l, c],
                send_sem=hs_sems.at[s, l, c],
                recv_sem=hr_sems.at[s, l, c],
                device_id=y_peer,
                device_id_type=pl.DeviceIdType.MESH,
            )
            h_rdmas[(s, l, c)] = r
            r.start()

        def o_send(s, l, d):
            r = pltpu.make_async_remote_copy(
                src_ref=osend_bf.at[s, l, d],
                dst_ref=orecv_bf.at[s, l, d],
                send_sem=os_sems.at[s, l, d],
                recv_sem=or_sems.at[s, l, d],
                device_id=x_peer,
                device_id_type=pl.DeviceIdType.MESH,
            )
            o_rdmas[(s, l, d)] = r
            r.start()

        def dot(a, b_val):
            return jnp.dot(a, b_val, preferred_element_type=jnp.float32)

        hvals = {}
        for s in range(S):
            xs = x_ref[s * bs:(s + 1) * bs, :]
            vals = []
            for c in range(C):
                if s == 0:
                    win0_copies[c].wait()
                v = dot(xs, win_scr[0, :, c * hc:(c + 1) * hc])
                vals.append(v)
                hsend_bf[s, 0, c] = v.astype(jnp.bfloat16)
                if s == 0 and c == 0:
                    pl.semaphore_wait(barrier, 2)
                h_send(s, 0, c)
            hvals[s] = vals

        for l in range(N_LAYERS):
            oaccs_by_s = {}
            for s in range(S):
                if s == 0:
                    w_copies[l][1].wait()
                oaccs = [None] * D
                for c in range(C):
                    h_rdmas[(s, l, c)].wait_recv()
                    hval = jnp.maximum(
                        hvals[s][c] + hrecv_bf[s, l, c].astype(jnp.float32),
                        0.0,
                    )
                    for d in range(D):
                        q = dot(
                            hval,
                            wout_scr[l, c * hc:(c + 1) * hc,
                                     d * oc:(d + 1) * oc],
                        )
                        oaccs[d] = q if oaccs[d] is None else oaccs[d] + q
                        if c == C - 1:
                            osend_bf[s, l, d] = oaccs[d].astype(jnp.bfloat16)
                            o_send(s, l, d)
                oaccs_by_s[s] = oaccs

            if l < N_LAYERS - 1:
                for s in range(S):
                    if s == 0:
                        w_copies[l + 1][0].wait()
                    g = [None] * C
                    for d in range(D):
                        o_rdmas[(s, l, d)].wait_recv()
                        cur_d = (
                            oaccs_by_s[s][d]
                            + orecv_bf[s, l, d].astype(jnp.float32)
                        )
                        for c in range(C):
                            q = dot(
                                cur_d,
                                win_scr[l + 1, d * oc:(d + 1) * oc,
                                        c * hc:(c + 1) * hc],
                            )
                            g[c] = q if g[c] is None else g[c] + q
                            if d == D - 1:
                                hsend_bf[s, l + 1, c] = g[c].astype(
                                    jnp.bfloat16
                                )
                                h_send(s, l + 1, c)
                    hvals[s] = g
            else:
                for s in range(S):
                    for d in range(D):
                        o_rdmas[(s, l, d)].wait_recv()
                        out_ref[s * bs:(s + 1) * bs,
                                d * oc:(d + 1) * oc] = (
                            oaccs_by_s[s][d]
                            + orecv_bf[s, l, d].astype(jnp.float32)
                        )

        for r in h_rdmas.values():
            r.wait_send()
        for r in o_rdmas.values():
            r.wait_send()

    return pl.pallas_call(
        body,
        out_shape=jax.ShapeDtypeStruct((b, o_per), jnp.float32),
        in_specs=[pl.BlockSpec(memory_space=pltpu.VMEM)]
        + [pl.BlockSpec(memory_space=pl.ANY)] * 6,
        out_specs=pl.BlockSpec(memory_space=pltpu.VMEM),
        scratch_shapes=[
            pltpu.VMEM((N_LAYERS, d_in, h_per), jnp.float32),
            pltpu.VMEM((N_LAYERS, h_per, o_per), jnp.float32),
            pltpu.VMEM((S, N_LAYERS, C, bs, hc), jnp.bfloat16),
            pltpu.VMEM((S, N_LAYERS, C, bs, hc), jnp.bfloat16),
            pltpu.VMEM((S, N_LAYERS, D, bs, oc), jnp.bfloat16),
            pltpu.VMEM((S, N_LAYERS, D, bs, oc), jnp.bfloat16),
            pltpu.SemaphoreType.DMA((2 * N_LAYERS,)),
            pltpu.SemaphoreType.DMA((C,)),
            pltpu.SemaphoreType.DMA((S, N_LAYERS, C)),
            pltpu.SemaphoreType.DMA((S, N_LAYERS, C)),
            pltpu.SemaphoreType.DMA((S, N_LAYERS, D)),
            pltpu.SemaphoreType.DMA((S, N_LAYERS, D)),
        ],
        compiler_params=pltpu.CompilerParams(
            collective_id=0,
            vmem_limit_bytes=100 * 1024 * 1024,
        ),
    )(x, Win0, Wout0, Win1, Wout1, Win2, Wout2)


# device time: 29450 ns/iter; 1.1038x vs baseline; 1.0016x over previous
import jax
import jax.numpy as jnp
from jax import lax
from jax.experimental import pallas as pl
from jax.experimental.pallas import tpu as pltpu

N_LAYERS = 3
S = 2
C = 4
D = 1


def kernel(x, Win0, Wout0, Win1, Wout1, Win2, Wout2):
    b, d_in = x.shape
    _, h_per = Win0.shape
    _, o_per = Wout0.shape
    bs = b // S
    hc = h_per // C
    oc = o_per // D

    def body(
        x_ref, win0_ref, wout0_ref, win1_ref, wout1_ref, win2_ref, wout2_ref,
        out_ref,
        win_scr, wout_scr,
        hsend_bf, hrecv_bf, osend_bf, orecv_bf,
        w_sems, w0_sems,
        hs_sems, hr_sems, os_sems, or_sems,
    ):
        my_x = lax.axis_index("x")
        my_y = lax.axis_index("y")
        y_peer = (my_x, 1 - my_y)
        x_peer = (1 - my_x, my_y)

        win_hbm = [win0_ref, win1_ref, win2_ref]
        wout_hbm = [wout0_ref, wout1_ref, wout2_ref]
        win0_copies = [
            pltpu.make_async_copy(
                win_hbm[0].at[:, c * hc:(c + 1) * hc],
                win_scr.at[0, :, c * hc:(c + 1) * hc],
                w0_sems.at[c],
            )
            for c in range(C)
        ]
        w_copies = [(None,
                     pltpu.make_async_copy(
                         wout_hbm[0], wout_scr.at[0], w_sems.at[1]))]
        for l in range(1, N_LAYERS):
            w_copies.append((
                pltpu.make_async_copy(
                    win_hbm[l], win_scr.at[l], w_sems.at[2 * l]),
                pltpu.make_async_copy(
                    wout_hbm[l], wout_scr.at[l], w_sems.at[2 * l + 1]),
            ))
        for cp in win0_copies:
            cp.start()
        w_copies[0][1].start()
        for cin, cout in w_copies[1:]:
            cin.start()
            cout.start()

        barrier = pltpu.get_barrier_semaphore()
        for nbr in (y_peer, x_peer):
            pl.semaphore_signal(
                barrier, inc=1, device_id=nbr,
                device_id_type=pl.DeviceIdType.MESH,
            )

        h_rdmas = {}
        o_rdmas = {}

        def h_send(s, l, c):
            r = pltpu.make_async_remote_copy(
                src_ref=hsend_bf.at[s, l, c],
                dst_ref=hrecv_bf.at[s, l, c],
                send_sem=hs_sems.at[s, l, c],
                recv_sem=hr_sems.at[s, l, c],
                device_id=y_peer,
                device_id_type=pl.DeviceIdType.MESH,
            )
            h_rdmas[(s, l, c)] = r
            r.start()

        def o_send(s, l, d):
            r = pltpu.make_async_remote_copy(
                src_ref=osend_bf.at[s, l, d],
                dst_ref=orecv_bf.at[s, l, d],
                send_sem=os_sems.at[s, l, d],
                recv_sem=or_sems.at[s, l, d],
                device_id=x_peer,
                device_id_type=pl.DeviceIdType.MESH,
            )
            o_rdmas[(s, l, d)] = r
            r.start()

        def dot(a, b_val):
            return jnp.dot(a, b_val, preferred_element_type=jnp.float32)

        hvals = {}
        for s in range(S):
            xs = x_ref[s * bs:(s + 1) * bs, :]
            vals = []
            for c in range(C):
                if s == 0:
                    win0_copies[c].wait()
                v = dot(xs, win_scr[0, :, c * hc:(c + 1) * hc])
                vals.append(v)
                hsend_bf[s, 0, c] = v.astype(jnp.bfloat16)
                if s == 0 and c == 0:
                    pl.semaphore_wait(barrier, 2)
                h_send(s, 0, c)
            hvals[s] = vals

        for l in range(N_LAYERS):
            oaccs_by_s = {}
            for s in range(S):
                if s == 0:
                    w_copies[l][1].wait()
                oaccs = [None] * D
                for c in range(C):
                    h_rdmas[(s, l, c)].wait_recv()
                    hval = jnp.maximum(
                        hvals[s][c] + hrecv_bf[s, l, c].astype(jnp.float32),
                        0.0,
                    )
                    for d in range(D):
                        q = dot(
                            hval,
                            wout_scr[l, c * hc:(c + 1) * hc,
                                     d * oc:(d + 1) * oc],
                        )
                        oaccs[d] = q if oaccs[d] is None else oaccs[d] + q
                        if c == C - 1:
                            osend_bf[s, l, d] = oaccs[d].astype(jnp.bfloat16)
                            o_send(s, l, d)
                oaccs_by_s[s] = oaccs

            if l < N_LAYERS - 1:
                for s in range(S):
                    if s == 0:
                        w_copies[l + 1][0].wait()
                    g = [None] * C
                    for d in range(D):
                        o_rdmas[(s, l, d)].wait_recv()
                        cur_d = (
                            oaccs_by_s[s][d]
                            + orecv_bf[s, l, d].astype(jnp.float32)
                        )
                        for c in range(C):
                            q = dot(
                                cur_d,
                                win_scr[l + 1, d * oc:(d + 1) * oc,
                                        c * hc:(c + 1) * hc],
                            )
                            g[c] = q if g[c] is None else g[c] + q
                            if d == D - 1:
                                hsend_bf[s, l + 1, c] = g[c].astype(
                                    jnp.bfloat16
                                )
                                h_send(s, l + 1, c)
                    hvals[s] = g
            else:
                for s in range(S):
                    for d in range(D):
                        o_rdmas[(s, l, d)].wait_recv()
                        out_ref[s * bs:(s + 1) * bs,
                                d * oc:(d + 1) * oc] = (
                            oaccs_by_s[s][d]
                            + orecv_bf[s, l, d].astype(jnp.float32)
                        )

        for r in h_rdmas.values():
            r.wait_send()
        for r in o_rdmas.values():
            r.wait_send()

    return pl.pallas_call(
        body,
        out_shape=jax.ShapeDtypeStruct((b, o_per), jnp.float32),
        in_specs=[pl.BlockSpec(memory_space=pltpu.VMEM)]
        + [pl.BlockSpec(memory_space=pl.ANY)] * 6,
        out_specs=pl.BlockSpec(memory_space=pltpu.VMEM),
        scratch_shapes=[
            pltpu.VMEM((N_LAYERS, d_in, h_per), jnp.float32),
            pltpu.VMEM((N_LAYERS, h_per, o_per), jnp.float32),
            pltpu.VMEM((S, N_LAYERS, C, bs, hc), jnp.bfloat16),
            pltpu.VMEM((S, N_LAYERS, C, bs, hc), jnp.bfloat16),
            pltpu.VMEM((S, N_LAYERS, D, bs, oc), jnp.bfloat16),
            pltpu.VMEM((S, N_LAYERS, D, bs, oc), jnp.bfloat16),
            pltpu.SemaphoreType.DMA((2 * N_LAYERS,)),
            pltpu.SemaphoreType.DMA((C,)),
            pltpu.SemaphoreType.DMA((S, N_LAYERS, C)),
            pltpu.SemaphoreType.DMA((S, N_LAYERS, C)),
            pltpu.SemaphoreType.DMA((S, N_LAYERS, D)),
            pltpu.SemaphoreType.DMA((S, N_LAYERS, D)),
        ],
        compiler_params=pltpu.CompilerParams(
            collective_id=0,
            vmem_limit_bytes=100 * 1024 * 1024,
        ),
    )(x, Win0, Wout0, Win1, Wout1, Win2, Wout2)
